# baseline (device time: 151864 ns/iter reference)
import jax
from jax import lax
from jax.experimental import pallas as pl
from jax.experimental.pallas import tpu as pltpu

N_DEV = 4
K = 4
H = N_DEV - 1


def kernel(x):
    m, n = x.shape
    mc = m // 2 // N_DEV
    msub = mc // K

    def subrows(half, c, j):
        return pl.ds(half * (N_DEV * mc) + c * mc + j * msub, msub)

    def rxsub(j):
        return pl.ds(j * msub, msub)

    def body(x_ref, out_ref, rx_p, rx_m,
             rs_ss_p, rs_rs_p, rs_ss_m, rs_rs_m,
             ag_ss_p, ag_rs_p, ag_ss_m, ag_rs_m):
        my = lax.axis_index("i")
        left = lax.rem(my + N_DEV - 1, N_DEV)
        right = lax.rem(my + 1, N_DEV)

        barrier_sem = pltpu.get_barrier_semaphore()
        for nbr in (left, right):
            pl.semaphore_signal(
                barrier_sem, inc=1,
                device_id=(nbr,), device_id_type=pl.DeviceIdType.MESH,
            )
        pl.semaphore_wait(barrier_sem, 2)

        def mk(src, dst, ssem, rsem, dev):
            return pltpu.make_async_remote_copy(
                src_ref=src, dst_ref=dst, send_sem=ssem, recv_sem=rsem,
                device_id=(dev,), device_id_type=pl.DeviceIdType.MESH,
            )

        rsd_p, rsd_m = {}, {}
        for j in range(K):
            d = mk(x_ref.at[subrows(0, my, j), :], rx_p.at[0, rxsub(j), :],
                   rs_ss_p.at[0, j], rs_rs_p.at[0, j], right)
            d.start(); rsd_p[(0, j)] = d
            d = mk(x_ref.at[subrows(1, my, j), :], rx_m.at[0, rxsub(j), :],
                   rs_ss_m.at[0, j], rs_rs_m.at[0, j], left)
            d.start(); rsd_m[(0, j)] = d

        for s in range(1, H):
            sp = lax.rem(my - s + N_DEV, N_DEV)
            sm = lax.rem(my + s, N_DEV)
            for j in range(K):
                rsd_p[(s - 1, j)].wait()
                out_ref[subrows(0, sp, j), :] = (
                    x_ref[subrows(0, sp, j), :] + rx_p[s - 1, rxsub(j), :]
                )
                d = mk(out_ref.at[subrows(0, sp, j), :],
                       rx_p.at[s, rxsub(j), :],
                       rs_ss_p.at[s, j], rs_rs_p.at[s, j], right)
                d.start(); rsd_p[(s, j)] = d
                rsd_m[(s - 1, j)].wait()
                out_ref[subrows(1, sm, j), :] = (
                    x_ref[subrows(1, sm, j), :] + rx_m[s - 1, rxsub(j), :]
                )
                d = mk(out_ref.at[subrows(1, sm, j), :],
                       rx_m.at[s, rxsub(j), :],
                       rs_ss_m.at[s, j], rs_rs_m.at[s, j], left)
                d.start(); rsd_m[(s, j)] = d

        fp = lax.rem(my + 1, N_DEV)
        fm = lax.rem(my + N_DEV - 1, N_DEV)
        agd_p, agd_m = {}, {}
        for j in range(K):
            rsd_p[(H - 1, j)].wait()
            out_ref[subrows(0, fp, j), :] = (
                x_ref[subrows(0, fp, j), :] + rx_p[H - 1, rxsub(j), :]
            )
            d = mk(out_ref.at[subrows(0, fp, j), :],
                   out_ref.at[subrows(0, fp, j), :],
                   ag_ss_p.at[0, j], ag_rs_p.at[0, j], right)
            d.start(); agd_p[(0, j)] = d
            rsd_m[(H - 1, j)].wait()
            out_ref[subrows(1, fm, j), :] = (
                x_ref[subrows(1, fm, j), :] + rx_m[H - 1, rxsub(j), :]
            )
            d = mk(out_ref.at[subrows(1, fm, j), :],
                   out_ref.at[subrows(1, fm, j), :],
                   ag_ss_m.at[0, j], ag_rs_m.at[0, j], left)
            d.start(); agd_m[(0, j)] = d

        for s in range(1, H):
            gp = lax.rem(my + 1 - s + N_DEV, N_DEV)
            gm = lax.rem(my - 1 + s + N_DEV, N_DEV)
            for j in range(K):
                agd_p[(s - 1, j)].wait()
                d = mk(out_ref.at[subrows(0, gp, j), :],
                       out_ref.at[subrows(0, gp, j), :],
                       ag_ss_p.at[s, j], ag_rs_p.at[s, j], right)
                d.start(); agd_p[(s, j)] = d
                agd_m[(s - 1, j)].wait()
                d = mk(out_ref.at[subrows(1, gm, j), :],
                       out_ref.at[subrows(1, gm, j), :],
                       ag_ss_m.at[s, j], ag_rs_m.at[s, j], left)
                d.start(); agd_m[(s, j)] = d

        for j in range(K):
            agd_p[(H - 1, j)].wait()
            agd_m[(H - 1, j)].wait()

    return pl.pallas_call(
        body,
        out_shape=jax.ShapeDtypeStruct((m, n), x.dtype),
        in_specs=[pl.BlockSpec(memory_space=pltpu.VMEM)],
        out_specs=pl.BlockSpec(memory_space=pltpu.VMEM),
        scratch_shapes=[
            pltpu.VMEM((H, mc, n), x.dtype),
            pltpu.VMEM((H, mc, n), x.dtype),
            pltpu.SemaphoreType.DMA((H, K)),
            pltpu.SemaphoreType.DMA((H, K)),
            pltpu.SemaphoreType.DMA((H, K)),
            pltpu.SemaphoreType.DMA((H, K)),
            pltpu.SemaphoreType.DMA((H, K)),
            pltpu.SemaphoreType.DMA((H, K)),
            pltpu.SemaphoreType.DMA((H, K)),
            pltpu.SemaphoreType.DMA((H, K)),
        ],
        compiler_params=pltpu.CompilerParams(collective_id=0),
    )(x)
